# baseline (speedup 1.0000x reference)
import jax
import jax.numpy as jnp
from jax.experimental import pallas as pl

_SEQ_BLOCK = 1024


def _add_kernel(x_ref, w_ref, o_ref):
    o_ref[...] = x_ref[...] + w_ref[...]


def kernel(x, weight):
    batch, seq, dim = x.shape
    w = weight[:seq, :]
    blk = _SEQ_BLOCK if seq % _SEQ_BLOCK == 0 else seq
    nblk = seq // blk
    x2 = x.reshape(batch * seq, dim)
    out2 = pl.pallas_call(
        _add_kernel,
        grid=(nblk, batch),
        in_specs=[
            pl.BlockSpec((blk, dim), lambda j, b: (b * nblk + j, 0)),
            pl.BlockSpec((blk, dim), lambda j, b: (j, 0)),
        ],
        out_specs=pl.BlockSpec((blk, dim), lambda j, b: (b * nblk + j, 0)),
        out_shape=jax.ShapeDtypeStruct((batch * seq, dim), x.dtype),
    )(x2, w)
    return out2.reshape(batch, seq, dim)


# speedup vs baseline: 1.0312x; 1.0312x over previous
import jax
import jax.numpy as jnp
from jax.experimental import pallas as pl

_SEQ_BLOCK = 256


def _add_kernel(x_ref, w_ref, o_ref):
    o_ref[...] = x_ref[...] + w_ref[...][None, :, :]


def kernel(x, weight):
    batch, seq, dim = x.shape
    w = weight[:seq, :]
    blk = _SEQ_BLOCK if seq % _SEQ_BLOCK == 0 else seq
    grid = (seq // blk,)
    return pl.pallas_call(
        _add_kernel,
        grid=grid,
        in_specs=[
            pl.BlockSpec((batch, blk, dim), lambda i: (0, i, 0)),
            pl.BlockSpec((blk, dim), lambda i: (i, 0)),
        ],
        out_specs=pl.BlockSpec((batch, blk, dim), lambda i: (0, i, 0)),
        out_shape=jax.ShapeDtypeStruct((batch, seq, dim), x.dtype),
    )(x, w)
